# Initial kernel scaffold; baseline (speedup 1.0000x reference)
import jax
import jax.numpy as jnp
from jax.experimental import pallas as pl


def kernel(x):
    raise NotImplementedError("write your pallas kernel here")


# speedup vs baseline: 3.0123x; 3.0123x over previous
import jax
import jax.numpy as jnp
from jax import lax
from jax.experimental import pallas as pl
from jax.experimental.pallas import tpu as pltpu
from jax.experimental.pallas import tpu_sc as plsc

ROWS = 128
COLS = 8192
KSEL = 64
L = 16
CH = COLS // L
NCORES = 2
NSUB = 16
NW = NCORES * NSUB
RPW = ROWS // NW

IMIN = -2147483648


def _keys_of(b):
    b = jnp.where(b == IMIN, jnp.int32(0), b)
    return jnp.where(b >= 0, b, ~(b ^ IMIN))


def _bits_of(k):
    return jnp.where(k >= 0, k, ~(k ^ IMIN))


def _kmax_body(x_hbm, out_hbm, key_v, hist, orow):
    wid = lax.axis_index("s") * NCORES + lax.axis_index("c")
    iota = lax.iota(jnp.int32, L)
    ones = jnp.ones((L,), jnp.int32)
    zeros = jnp.zeros((L,), jnp.int32)

    def zero_hist():
        for j in range(16):
            hist[pl.ds(16 * j, 16)] = zeros

    def scan_level(quota):
        acc = zeros
        for j in range(16):
            hv = hist[pl.ds(16 * j, 16)]
            acc = jnp.where(iota == j, jnp.sum(hv), acc)
        suf_g = jnp.flip(plsc.cumsum(jnp.flip(acc)))
        g = plsc.all_reduce_population_count(suf_g >= quota) - 1
        after = jnp.sum(jnp.where(iota == g + 1, suf_g, 0))
        hv = plsc.load_gather(hist, [g * 16 + iota])
        suf_in = jnp.flip(plsc.cumsum(jnp.flip(hv))) + after
        lane = plsc.all_reduce_population_count(suf_in >= quota) - 1
        b = g * 16 + lane
        s1 = jnp.sum(jnp.where(iota == lane + 1, suf_in, 0))
        lane_s = jnp.max(lane)
        cnt_gt = jnp.where(lane_s == jnp.int32(15), after, s1)
        return jnp.max(b), cnt_gt

    def do_row(r):
        pltpu.sync_copy(x_hbm.at[r], key_v)

        zero_hist()

        def p0(i, c):
            b = key_v[pl.ds(i * L, L)]
            key = _keys_of(b)
            key_v[pl.ds(i * L, L)] = key
            digit = lax.shift_right_logical(key, 24) ^ 0x80
            plsc.addupdate_scatter(hist, [digit], ones)
            return c

        lax.fori_loop(0, CH, p0, 0)

        quota = jnp.int32(KSEL)
        b, cnt_gt = scan_level(quota)
        quota = quota - cnt_gt
        prefix = (b ^ 0x80) << 24

        for lvl in (2, 1, 0):
            zero_hist()
            hi_shift = 8 * (lvl + 1)
            pref_hi = lax.shift_right_logical(prefix, hi_shift)

            def hp(i, c, hi_shift=hi_shift, lvl=lvl, pref_hi=pref_hi):
                k = key_v[pl.ds(i * L, L)]
                m = lax.shift_right_logical(k, hi_shift) == pref_hi
                digit = lax.shift_right_logical(k, 8 * lvl) & 0xFF
                plsc.addupdate_scatter(hist, [digit], ones, mask=m)
                return c

            lax.fori_loop(0, CH, hp, 0)
            b, cnt_gt = scan_level(quota)
            quota = quota - cnt_gt
            prefix = prefix | (b << (8 * lvl))

        thresh = prefix

        def sel(i, carry):
            selc, eqc = carry
            k = key_v[pl.ds(i * L, L)]
            m_gt = k > thresh
            m_eq = k == thresh
            eqv = m_eq.astype(jnp.int32)
            eq_rank = eqc + plsc.cumsum(eqv)
            m_sel = m_gt | (m_eq & (eq_rank <= quota))
            si = m_sel.astype(jnp.int32)
            pos = selc + plsc.cumsum(si) - 1
            plsc.store_scatter(orow, [pos], _bits_of(k), mask=m_sel)
            return selc + jnp.sum(si), eqc + jnp.sum(eqv)

        lax.fori_loop(0, CH, sel, (jnp.int32(0), jnp.int32(0)))
        pltpu.sync_copy(orow, out_hbm.at[r])

    for j in range(RPW):
        do_row(wid * RPW + j)


_SCRATCH = [
    pltpu.VMEM((COLS,), jnp.int32),
    pltpu.VMEM((256,), jnp.int32),
    pltpu.VMEM((KSEL,), jnp.int32),
]

_kmax_rows = pl.kernel(
    _kmax_body,
    out_type=jax.ShapeDtypeStruct((ROWS, KSEL), jnp.int32),
    mesh=plsc.VectorSubcoreMesh(
        core_axis_name="c", subcore_axis_name="s",
        num_cores=NCORES, num_subcores=NSUB,
    ),
    scratch_types=_SCRATCH,
    compiler_params=pltpu.CompilerParams(needs_layout_passes=False),
)


def kernel(x):
    bits = lax.bitcast_convert_type(x, jnp.int32)
    out = _kmax_rows(bits)
    return lax.bitcast_convert_type(out, jnp.float32)


# speedup vs baseline: 5.2199x; 1.7329x over previous
import jax
import jax.numpy as jnp
from jax import lax
from jax.experimental import pallas as pl
from jax.experimental.pallas import tpu as pltpu
from jax.experimental.pallas import tpu_sc as plsc

ROWS = 128
COLS = 8192
KSEL = 64
L = 16
CH = COLS // L
NCORES = 2
NSUB = 16
NW = NCORES * NSUB
RPW = ROWS // NW

IMIN = -2147483648


def _keys_of(b):
    b = jnp.where(b == IMIN, jnp.int32(0), b)
    return jnp.where(b >= 0, b, ~(b ^ IMIN))


def _bits_of(k):
    return jnp.where(k >= 0, k, ~(k ^ IMIN))


def _kmax_body(x_hbm, out_hbm, key_v, cand_v, hist, orow):
    wid = lax.axis_index("s") * NCORES + lax.axis_index("c")
    iota = lax.iota(jnp.int32, L)
    ones = jnp.ones((L,), jnp.int32)
    zeros = jnp.zeros((L,), jnp.int32)

    def zero_hist():
        for j in range(16):
            hist[pl.ds(16 * j, 16)] = zeros

    def scan_level(quota):
        acc = zeros
        for j in range(16):
            hv = hist[pl.ds(16 * j, 16)]
            acc = jnp.where(iota == j, jnp.sum(hv), acc)
        suf_g = jnp.flip(plsc.cumsum(jnp.flip(acc)))
        g = plsc.all_reduce_population_count(suf_g >= quota) - 1
        after = jnp.sum(jnp.where(iota == g + 1, suf_g, 0))
        hv = plsc.load_gather(hist, [g * 16 + iota])
        suf_in = jnp.flip(plsc.cumsum(jnp.flip(hv))) + after
        lane = plsc.all_reduce_population_count(suf_in >= quota) - 1
        b = g * 16 + lane
        s1 = jnp.sum(jnp.where(iota == lane + 1, suf_in, 0))
        lane_s = jnp.max(lane)
        cnt_gt = jnp.where(lane_s == jnp.int32(15), after, s1)
        return jnp.max(b), cnt_gt

    def do_row(r):
        pltpu.sync_copy(x_hbm.at[r], key_v)

        zero_hist()

        def p0(i, c):
            b = key_v[pl.ds(i * L, L)]
            key = _keys_of(b)
            key_v[pl.ds(i * L, L)] = key
            digit = lax.shift_right_logical(key, 24) ^ 0x80
            plsc.addupdate_scatter(hist, [digit], ones)
            return c

        lax.fori_loop(0, CH, p0, 0)

        quota = jnp.int32(KSEL)
        b, cnt_gt = scan_level(quota)
        quota = quota - cnt_gt
        prefix = (b ^ 0x80) << 24

        zero_hist()
        pref3 = lax.shift_right_arithmetic(prefix, 24)

        def p1(i, nc):
            k = key_v[pl.ds(i * L, L)]
            top = lax.shift_right_arithmetic(k, 24)
            m = top >= pref3
            m_eq3 = top == pref3
            pos = nc + plsc.cumsum(m.astype(jnp.int32)) - 1
            plsc.store_scatter(cand_v, [pos], k, mask=m)
            digit = lax.shift_right_logical(k, 16) & 0xFF
            plsc.addupdate_scatter(hist, [digit], ones, mask=m_eq3)
            return nc + plsc.all_reduce_population_count(m)

        n_c = lax.fori_loop(0, CH, p1, zeros)
        nv = lax.shift_right_logical(jnp.max(n_c) + (L - 1), 4)

        b, cnt_gt = scan_level(quota)
        quota = quota - cnt_gt
        prefix = prefix | (b << 16)

        for lvl in (1, 0):
            zero_hist()
            hi_shift = 8 * (lvl + 1)
            pref_hi = lax.shift_right_logical(prefix, hi_shift)

            def hp(i, c, hi_shift=hi_shift, lvl=lvl, pref_hi=pref_hi):
                k = cand_v[pl.ds(i * L, L)]
                valid = (i * L + iota) < n_c
                m = (lax.shift_right_logical(k, hi_shift) == pref_hi) & valid
                digit = lax.shift_right_logical(k, 8 * lvl) & 0xFF
                plsc.addupdate_scatter(hist, [digit], ones, mask=m)
                return c

            lax.fori_loop(0, nv, hp, 0)
            b, cnt_gt = scan_level(quota)
            quota = quota - cnt_gt
            prefix = prefix | (b << (8 * lvl))

        thresh = prefix

        def sel(i, carry):
            selc, eqc = carry
            k = cand_v[pl.ds(i * L, L)]
            valid = (i * L + iota) < n_c
            m_gt = (k > thresh) & valid
            m_eq = (k == thresh) & valid
            eq_rank = eqc + plsc.cumsum(m_eq.astype(jnp.int32))
            m_sel = m_gt | (m_eq & (eq_rank <= quota))
            pos = selc + plsc.cumsum(m_sel.astype(jnp.int32)) - 1
            plsc.store_scatter(orow, [pos], _bits_of(k), mask=m_sel)
            return (selc + plsc.all_reduce_population_count(m_sel),
                    eqc + plsc.all_reduce_population_count(m_eq))

        lax.fori_loop(0, nv, sel, (zeros, zeros))
        pltpu.sync_copy(orow, out_hbm.at[r])

    for j in range(RPW):
        do_row(wid * RPW + j)


_SCRATCH = [
    pltpu.VMEM((COLS,), jnp.int32),
    pltpu.VMEM((COLS,), jnp.int32),
    pltpu.VMEM((256,), jnp.int32),
    pltpu.VMEM((KSEL,), jnp.int32),
]

_kmax_rows = pl.kernel(
    _kmax_body,
    out_type=jax.ShapeDtypeStruct((ROWS, KSEL), jnp.int32),
    mesh=plsc.VectorSubcoreMesh(
        core_axis_name="c", subcore_axis_name="s",
        num_cores=NCORES, num_subcores=NSUB,
    ),
    scratch_types=_SCRATCH,
    compiler_params=pltpu.CompilerParams(needs_layout_passes=False),
)


def kernel(x):
    bits = lax.bitcast_convert_type(x, jnp.int32)
    out = _kmax_rows(bits)
    return lax.bitcast_convert_type(out, jnp.float32)
